# baseline (speedup 1.0000x reference)
import functools

import jax
import jax.numpy as jnp
from jax import lax
from jax.experimental import pallas as pl
from jax.experimental.pallas import tpu as pltpu
from jax.experimental.pallas import tpu_sc as plsc

_NC = 2
_NS = 16
_CHUNK = 128
_G = 16
_BLK = 1000


def _pre_block(x_ref, wt_ref, wb_ref, g2_ref, b2_ref, lf_ref, xi_ref,
               y_ref, i1_ref):
    h = jnp.dot(x_ref[...], wt_ref[...], preferred_element_type=jnp.float32)
    h = h + wb_ref[...]
    m = jnp.mean(h, axis=-1, keepdims=True)
    d = h - m
    v = jnp.mean(d * d, axis=-1, keepdims=True)
    hn = d * lax.rsqrt(v + 1e-5)
    y_ref[...] = hn * g2_ref[...] + b2_ref[...]
    i1_ref[...] = xi_ref[0, 0] * lf_ref[...]


def _post_block(s0_ref, s1_ref, owt_ref, ob_ref, out_ref):
    u = s0_ref[...] + s1_ref[...]
    u = jnp.maximum(u, 0.0)
    out_ref[...] = (jnp.dot(u, owt_ref[...], preferred_element_type=jnp.float32)
                    + ob_ref[...] + u)


@functools.lru_cache(maxsize=None)
def _pre_call(n, d):
    grid = (n // _BLK,)
    row_spec = pl.BlockSpec((_BLK, d), lambda i: (i, 0))
    vec_spec = pl.BlockSpec((1, d), lambda i: (0, 0))
    return pl.pallas_call(
        _pre_block,
        grid=grid,
        in_specs=[
            row_spec,
            pl.BlockSpec((d, d), lambda i: (0, 0)),
            vec_spec,
            vec_spec,
            vec_spec,
            row_spec,
            pl.BlockSpec((1, 1), lambda i: (0, 0)),
        ],
        out_specs=[row_spec, row_spec],
        out_shape=[jax.ShapeDtypeStruct((n, d), jnp.float32)] * 2,
    )


@functools.lru_cache(maxsize=None)
def _post_call(n, d):
    grid = (n // _BLK,)
    row_spec = pl.BlockSpec((_BLK, d), lambda i: (i, 0))
    vec_spec = pl.BlockSpec((1, d), lambda i: (0, 0))
    return pl.pallas_call(
        _post_block,
        grid=grid,
        in_specs=[
            row_spec,
            row_spec,
            pl.BlockSpec((d, d), lambda i: (0, 0)),
            vec_spec,
        ],
        out_specs=row_spec,
        out_shape=jax.ShapeDtypeStruct((n, d), jnp.float32),
    )


@functools.lru_cache(maxsize=None)
def _sc_accum(n, d, k):
    rpt = -(-n // _NS) // 8 * 8 + 8
    rpt_last = n - (_NS - 1) * rpt
    assert rpt % 8 == 0 and rpt_last > 0 and rpt_last % 8 == 0
    acc_rows = n + _CHUNK
    mesh = plsc.VectorSubcoreMesh(core_axis_name="c", subcore_axis_name="s")

    @functools.partial(
        pl.kernel, mesh=mesh,
        out_type=jax.ShapeDtypeStruct((_NC * n, d), jnp.float32),
        scratch_types=[
            pltpu.VMEM((_G, _CHUNK), jnp.int32),
            pltpu.VMEM((_G, _CHUNK), jnp.int32),
            pltpu.VMEM((_CHUNK, d), jnp.float32),
            pltpu.VMEM((_CHUNK, d), jnp.float32),
            pltpu.VMEM_SHARED((acc_rows, d), jnp.float32),
            pltpu.SemaphoreType.DMA,
            pltpu.SemaphoreType.DMA,
            pltpu.SemaphoreType.DMA,
            pltpu.SemaphoreType.DMA,
        ],
    )
    def body(y_hbm, x_hbm, i1_hbm, src_hbm, dst_hbm, out_hbm,
             src_v, dst_v, rows0, rows1, s_sh, gsem0, gsem1, ssem0, ssem1):
        cid = lax.axis_index("c")
        sid = lax.axis_index("s")
        wid = sid * _NC + cid
        base = sid * rpt

        def drain(buf, sem):
            pltpu.make_async_copy(y_hbm.at[pl.ds(0, _CHUNK)], buf, sem).wait()

        def seed(rows):
            sl = pl.ds(base, rows)

            @pl.when(cid == 0)
            def _():
                pltpu.sync_copy(x_hbm.at[sl], s_sh.at[sl])

            @pl.when(cid != 0)
            def _():
                pltpu.sync_copy(i1_hbm.at[sl], s_sh.at[sl])

        @pl.when(sid < _NS - 1)
        def _():
            seed(rpt)

        @pl.when(sid == _NS - 1)
        def _():
            seed(rpt_last)

        plsc.subcore_barrier()

        def group(g, carry):
            gbase = g * _G
            pltpu.sync_copy(src_hbm.at[wid, pl.ds(gbase, _G)], src_v)
            pltpu.sync_copy(dst_hbm.at[wid, pl.ds(gbase, _G)], dst_v)
            pltpu.async_copy(y_hbm.at[src_v.at[0]], rows0, gsem0)
            pltpu.async_copy(y_hbm.at[src_v.at[1]], rows1, gsem1)

            def step(i, c2):
                j0 = i * 2
                j1 = j0 + 1
                drain(rows0, gsem0)
                pltpu.async_copy(rows0, s_sh.at[dst_v.at[j0]], ssem0,
                                 add=True)
                drain(rows1, gsem1)
                pltpu.async_copy(rows1, s_sh.at[dst_v.at[j1]], ssem1,
                                 add=True)
                drain(rows0, ssem0)

                @pl.when(j0 + 2 < _G)
                def _():
                    pltpu.async_copy(y_hbm.at[src_v.at[j0 + 2]], rows0, gsem0)

                drain(rows1, ssem1)

                @pl.when(j1 + 2 < _G)
                def _():
                    pltpu.async_copy(y_hbm.at[src_v.at[j1 + 2]], rows1, gsem1)

                return c2

            lax.fori_loop(0, _G // 2, step, 0)
            return carry

        lax.fori_loop(0, k // _G, group, 0)
        plsc.subcore_barrier()

        def wb(rows):
            sl = pl.ds(base, rows)
            pltpu.sync_copy(s_sh.at[sl], out_hbm.at[pl.ds(cid * n + base,
                                                          rows)])

        @pl.when(sid < _NS - 1)
        def _():
            wb(rpt)

        @pl.when(sid == _NS - 1)
        def _():
            wb(rpt_last)

    return body


def kernel(x, edge_index, edge_attr, local_features, timestep,
           gamma, eta, xi, W_w, W_b, ln_g, ln_b, out_w, out_b):
    n, d = x.shape
    e = edge_index.shape[1]
    nw = _NC * _NS
    k = -(-e // (nw * _CHUNK * _G)) * _G
    e_pad = nw * k * _CHUNK

    gamma_t = jax.nn.sigmoid(gamma[timestep])
    eta_t = jax.nn.sigmoid(eta[timestep])
    c = gamma_t * (1.0 - eta_t)
    g2 = (c * ln_g).reshape(1, d)
    b2 = (c * ln_b).reshape(1, d)

    src = edge_index[0]
    dst = edge_index[1]
    pad = e_pad - e
    src_p = jnp.concatenate(
        [src, jnp.zeros((pad,), jnp.int32)]).reshape(nw, k, _CHUNK)
    dst_p = jnp.concatenate(
        [dst, n + jnp.arange(pad, dtype=jnp.int32) % _CHUNK]
    ).reshape(nw, k, _CHUNK)

    y, init1 = _pre_call(n, d)(
        x, W_w.T, W_b.reshape(1, d), g2, b2,
        local_features, xi.reshape(1, 1))

    s = _sc_accum(n, d, k)(y, x, init1, src_p, dst_p)

    return _post_call(n, d)(
        s[:n], s[n:], out_w.T, out_b.reshape(1, d))


# speedup vs baseline: 1.1922x; 1.1922x over previous
import functools

import jax
import jax.numpy as jnp
from jax import lax
from jax.experimental import pallas as pl
from jax.experimental.pallas import tpu as pltpu
from jax.experimental.pallas import tpu_sc as plsc

_NC = 2
_NS = 16
_CHUNK = 128
_G = 16
_BLK = 1000


def _pre_block(x_ref, wt_ref, wb_ref, g2_ref, b2_ref, lf_ref, xi_ref,
               y_ref, i1_ref):
    h = jnp.dot(x_ref[...], wt_ref[...], preferred_element_type=jnp.float32)
    h = h + wb_ref[...]
    m = jnp.mean(h, axis=-1, keepdims=True)
    d = h - m
    v = jnp.mean(d * d, axis=-1, keepdims=True)
    hn = d * lax.rsqrt(v + 1e-5)
    y_ref[...] = hn * g2_ref[...] + b2_ref[...]
    i1_ref[...] = xi_ref[0, 0] * lf_ref[...]


def _post_block(s0_ref, s1_ref, owt_ref, ob_ref, out_ref):
    u = s0_ref[...] + s1_ref[...]
    u = jnp.maximum(u, 0.0)
    out_ref[...] = (jnp.dot(u, owt_ref[...], preferred_element_type=jnp.float32)
                    + ob_ref[...] + u)


@functools.lru_cache(maxsize=None)
def _pre_call(n, d):
    grid = (n // _BLK,)
    row_spec = pl.BlockSpec((_BLK, d), lambda i: (i, 0))
    vec_spec = pl.BlockSpec((1, d), lambda i: (0, 0))
    return pl.pallas_call(
        _pre_block,
        grid=grid,
        in_specs=[
            row_spec,
            pl.BlockSpec((d, d), lambda i: (0, 0)),
            vec_spec,
            vec_spec,
            vec_spec,
            row_spec,
            pl.BlockSpec((1, 1), lambda i: (0, 0)),
        ],
        out_specs=[row_spec, row_spec],
        out_shape=[jax.ShapeDtypeStruct((n, d), jnp.float32)] * 2,
    )


@functools.lru_cache(maxsize=None)
def _post_call(n, d):
    grid = (n // _BLK,)
    row_spec = pl.BlockSpec((_BLK, d), lambda i: (i, 0))
    vec_spec = pl.BlockSpec((1, d), lambda i: (0, 0))
    return pl.pallas_call(
        _post_block,
        grid=grid,
        in_specs=[
            row_spec,
            row_spec,
            pl.BlockSpec((d, d), lambda i: (0, 0)),
            vec_spec,
        ],
        out_specs=row_spec,
        out_shape=jax.ShapeDtypeStruct((n, d), jnp.float32),
    )


@functools.lru_cache(maxsize=None)
def _sc_accum(n, d, k):
    rpt = -(-n // _NS) // 8 * 8 + 8
    rpt_last = n - (_NS - 1) * rpt
    assert rpt % 8 == 0 and rpt_last > 0 and rpt_last % 8 == 0
    acc_rows = n + _CHUNK
    mesh = plsc.VectorSubcoreMesh(core_axis_name="c", subcore_axis_name="s")

    @functools.partial(
        pl.kernel, mesh=mesh,
        out_type=jax.ShapeDtypeStruct((_NC * n, d), jnp.float32),
        scratch_types=[
            pltpu.VMEM((_G, _CHUNK), jnp.int32),
            pltpu.VMEM((_G, _CHUNK), jnp.int32),
            pltpu.VMEM((_CHUNK, d), jnp.float32),
            pltpu.VMEM((_CHUNK, d), jnp.float32),
            pltpu.VMEM_SHARED((acc_rows, d), jnp.float32),
            pltpu.SemaphoreType.DMA,
            pltpu.SemaphoreType.DMA,
        ],
    )
    def body(y_hbm, x_hbm, i1_hbm, src_hbm, dst_hbm, out_hbm,
             src_v, dst_v, rows0, rows1, s_sh, gsem0, gsem1):
        cid = lax.axis_index("c")
        sid = lax.axis_index("s")
        wid = sid * _NC + cid
        base = sid * rpt

        def drain(buf, sem):
            pltpu.make_async_copy(y_hbm.at[pl.ds(0, _CHUNK)], buf, sem).wait()

        def seed(rows):
            sl = pl.ds(base, rows)

            @pl.when(cid == 0)
            def _():
                pltpu.sync_copy(x_hbm.at[sl], s_sh.at[sl])

            @pl.when(cid != 0)
            def _():
                pltpu.sync_copy(i1_hbm.at[sl], s_sh.at[sl])

        @pl.when(sid < _NS - 1)
        def _():
            seed(rpt)

        @pl.when(sid == _NS - 1)
        def _():
            seed(rpt_last)

        plsc.subcore_barrier()

        def group(g, carry):
            gbase = g * _G
            pltpu.sync_copy(src_hbm.at[wid, pl.ds(gbase, _G)], src_v)
            pltpu.sync_copy(dst_hbm.at[wid, pl.ds(gbase, _G)], dst_v)
            pltpu.async_copy(y_hbm.at[src_v.at[0]], rows0, gsem0)

            def step(i, c2):
                j0 = i * 2
                j1 = j0 + 1
                pltpu.async_copy(y_hbm.at[src_v.at[j1]], rows1, gsem1)
                drain(rows0, gsem0)
                pltpu.sync_copy(rows0, s_sh.at[dst_v.at[j0]], add=True)

                @pl.when(j0 + 2 < _G)
                def _():
                    pltpu.async_copy(y_hbm.at[src_v.at[j0 + 2]], rows0, gsem0)

                drain(rows1, gsem1)
                pltpu.sync_copy(rows1, s_sh.at[dst_v.at[j1]], add=True)
                return c2

            lax.fori_loop(0, _G // 2, step, 0)
            return carry

        lax.fori_loop(0, k // _G, group, 0)
        plsc.subcore_barrier()

        def wb(rows):
            sl = pl.ds(base, rows)
            pltpu.sync_copy(s_sh.at[sl], out_hbm.at[pl.ds(cid * n + base,
                                                          rows)])

        @pl.when(sid < _NS - 1)
        def _():
            wb(rpt)

        @pl.when(sid == _NS - 1)
        def _():
            wb(rpt_last)

    return body


def kernel(x, edge_index, edge_attr, local_features, timestep,
           gamma, eta, xi, W_w, W_b, ln_g, ln_b, out_w, out_b):
    n, d = x.shape
    e = edge_index.shape[1]
    nw = _NC * _NS
    k = -(-e // (nw * _CHUNK * _G)) * _G
    e_pad = nw * k * _CHUNK

    gamma_t = jax.nn.sigmoid(gamma[timestep])
    eta_t = jax.nn.sigmoid(eta[timestep])
    c = gamma_t * (1.0 - eta_t)
    g2 = (c * ln_g).reshape(1, d)
    b2 = (c * ln_b).reshape(1, d)

    src = edge_index[0]
    dst = edge_index[1]
    pad = e_pad - e
    src_p = jnp.concatenate(
        [src, jnp.zeros((pad,), jnp.int32)]
    ).reshape(k, nw, _CHUNK).transpose(1, 0, 2)
    dst_p = jnp.concatenate(
        [dst, n + jnp.arange(pad, dtype=jnp.int32) % _CHUNK]
    ).reshape(k, nw, _CHUNK).transpose(1, 0, 2)

    y, init1 = _pre_call(n, d)(
        x, W_w.T, W_b.reshape(1, d), g2, b2,
        local_features, xi.reshape(1, 1))

    s = _sc_accum(n, d, k)(y, x, init1, src_p, dst_p)

    return _post_call(n, d)(
        s[:n], s[n:], out_w.T, out_b.reshape(1, d))


# speedup vs baseline: 2.8099x; 2.3569x over previous
import functools

import jax
import jax.numpy as jnp
from jax import lax
from jax.experimental import pallas as pl
from jax.experimental.pallas import tpu as pltpu
from jax.experimental.pallas import tpu_sc as plsc

_NC = 2
_NS = 16
_CHUNK = 128
_G = 16
_BLK = 1000


def _pre_block(x_ref, wt_ref, wb_ref, g2_ref, b2_ref, lf_ref, xi_ref,
               y_ref, i1_ref):
    h = jnp.dot(x_ref[...], wt_ref[...], preferred_element_type=jnp.float32)
    h = h + wb_ref[...]
    m = jnp.mean(h, axis=-1, keepdims=True)
    d = h - m
    v = jnp.mean(d * d, axis=-1, keepdims=True)
    hn = d * lax.rsqrt(v + 1e-5)
    y_ref[...] = hn * g2_ref[...] + b2_ref[...]
    i1_ref[...] = xi_ref[0, 0] * lf_ref[...]


def _post_block(s0_ref, s1_ref, owt_ref, ob_ref, out_ref):
    u = s0_ref[...] + s1_ref[...]
    u = jnp.maximum(u, 0.0)
    out_ref[...] = (jnp.dot(u, owt_ref[...], preferred_element_type=jnp.float32)
                    + ob_ref[...] + u)


@functools.lru_cache(maxsize=None)
def _pre_call(n, d):
    grid = (n // _BLK,)
    row_spec = pl.BlockSpec((_BLK, d), lambda i: (i, 0))
    vec_spec = pl.BlockSpec((1, d), lambda i: (0, 0))
    return pl.pallas_call(
        _pre_block,
        grid=grid,
        in_specs=[
            row_spec,
            pl.BlockSpec((d, d), lambda i: (0, 0)),
            vec_spec,
            vec_spec,
            vec_spec,
            row_spec,
            pl.BlockSpec((1, 1), lambda i: (0, 0)),
        ],
        out_specs=[row_spec, row_spec],
        out_shape=[jax.ShapeDtypeStruct((n, d), jnp.float32)] * 2,
    )


@functools.lru_cache(maxsize=None)
def _post_call(n, d):
    grid = (n // _BLK,)
    row_spec = pl.BlockSpec((_BLK, d), lambda i: (i, 0))
    vec_spec = pl.BlockSpec((1, d), lambda i: (0, 0))
    return pl.pallas_call(
        _post_block,
        grid=grid,
        in_specs=[
            row_spec,
            row_spec,
            pl.BlockSpec((d, d), lambda i: (0, 0)),
            vec_spec,
        ],
        out_specs=row_spec,
        out_shape=jax.ShapeDtypeStruct((n, d), jnp.float32),
    )


@functools.lru_cache(maxsize=None)
def _sc_accum(n, d, k):
    rpt = -(-n // _NS) // 8 * 8 + 8
    rpt_last = n - (_NS - 1) * rpt
    assert rpt % 8 == 0 and rpt_last > 0 and rpt_last % 8 == 0
    acc_rows = n + 64 * _NS
    mesh = plsc.VectorSubcoreMesh(core_axis_name="c", subcore_axis_name="s")

    @functools.partial(
        pl.kernel, mesh=mesh,
        out_type=jax.ShapeDtypeStruct((_NC * n, d), jnp.float32),
        scratch_types=[
            pltpu.VMEM((_G, _CHUNK), jnp.int32),
            pltpu.VMEM((_G, _CHUNK), jnp.int32),
            pltpu.VMEM((_CHUNK, d), jnp.float32),
            pltpu.VMEM((_CHUNK, d), jnp.float32),
            pltpu.VMEM_SHARED((acc_rows, d), jnp.float32),
            pltpu.SemaphoreType.DMA,
            pltpu.SemaphoreType.DMA,
        ],
    )
    def body(y_hbm, x_hbm, i1_hbm, src_hbm, dst_hbm, out_hbm,
             src_v, dst_v, rows0, rows1, s_sh, gsem0, gsem1):
        cid = lax.axis_index("c")
        sid = lax.axis_index("s")
        wid = sid * _NC + cid
        base = sid * rpt

        def drain(buf, sem):
            pltpu.make_async_copy(y_hbm.at[pl.ds(0, _CHUNK)], buf, sem).wait()

        def seed(rows):
            sl = pl.ds(base, rows)

            @pl.when(cid == 0)
            def _():
                pltpu.sync_copy(x_hbm.at[sl], s_sh.at[sl])

            @pl.when(cid != 0)
            def _():
                pltpu.sync_copy(i1_hbm.at[sl], s_sh.at[sl])

        @pl.when(sid < _NS - 1)
        def _():
            seed(rpt)

        @pl.when(sid == _NS - 1)
        def _():
            seed(rpt_last)

        plsc.subcore_barrier()

        def group(g, carry):
            gbase = g * _G
            pltpu.sync_copy(src_hbm.at[wid, pl.ds(gbase, _G)], src_v)
            pltpu.sync_copy(dst_hbm.at[wid, pl.ds(gbase, _G)], dst_v)
            pltpu.async_copy(y_hbm.at[src_v.at[0]], rows0, gsem0)

            def step(i, c2):
                j0 = i * 2
                j1 = j0 + 1
                pltpu.async_copy(y_hbm.at[src_v.at[j1]], rows1, gsem1)
                drain(rows0, gsem0)
                pltpu.sync_copy(rows0, s_sh.at[dst_v.at[j0]], add=True)

                @pl.when(j0 + 2 < _G)
                def _():
                    pltpu.async_copy(y_hbm.at[src_v.at[j0 + 2]], rows0, gsem0)

                drain(rows1, gsem1)
                pltpu.sync_copy(rows1, s_sh.at[dst_v.at[j1]], add=True)
                return c2

            lax.fori_loop(0, _G // 2, step, 0)
            return carry

        lax.fori_loop(0, k // _G, group, 0)
        plsc.subcore_barrier()

        def wb(rows):
            sl = pl.ds(base, rows)
            pltpu.sync_copy(s_sh.at[sl], out_hbm.at[pl.ds(cid * n + base,
                                                          rows)])

        @pl.when(sid < _NS - 1)
        def _():
            wb(rpt)

        @pl.when(sid == _NS - 1)
        def _():
            wb(rpt_last)

    return body


def kernel(x, edge_index, edge_attr, local_features, timestep,
           gamma, eta, xi, W_w, W_b, ln_g, ln_b, out_w, out_b):
    n, d = x.shape
    e = edge_index.shape[1]
    nw = _NC * _NS
    k = -(-e // (nw * _CHUNK * _G)) * _G
    e_pad = nw * k * _CHUNK

    gamma_t = jax.nn.sigmoid(gamma[timestep])
    eta_t = jax.nn.sigmoid(eta[timestep])
    c = gamma_t * (1.0 - eta_t)
    g2 = (c * ln_g).reshape(1, d)
    b2 = (c * ln_b).reshape(1, d)

    src = edge_index[0]
    dst = edge_index[1]
    pad = e_pad - e
    k_main = (e // (nw * _CHUNK)) // 2 * 2
    e_main = nw * k_main * _CHUNK
    k_tail = k - k_main
    t_len = nw * k_tail * _CHUNK
    src_main = src[:e_main].reshape(nw, k_main, _CHUNK)
    dst_main = dst[:e_main].reshape(nw, k_main, _CHUNK)
    pos = jnp.arange(t_len, dtype=jnp.int32)
    n_real_tail = e - e_main
    is_pad = pos >= n_real_tail
    src_tail = jnp.where(
        is_pad, pos % jnp.int32(n),
        jnp.concatenate([src[e_main:], jnp.zeros((pad,), jnp.int32)]))
    w_idx = pos.reshape(nw, k_tail, _CHUNK) // (k_tail * _CHUNK)
    trash = n + (w_idx // _NC) * 64 + pos.reshape(nw, k_tail, _CHUNK) % 64
    dst_tail = jnp.where(
        is_pad.reshape(nw, k_tail, _CHUNK), trash,
        jnp.concatenate([dst[e_main:], jnp.zeros((pad,), jnp.int32)]
                        ).reshape(nw, k_tail, _CHUNK))
    src_p = jnp.concatenate(
        [src_main, src_tail.reshape(nw, k_tail, _CHUNK)], axis=1)
    dst_p = jnp.concatenate([dst_main, dst_tail], axis=1)

    y, init1 = _pre_call(n, d)(
        x, W_w.T, W_b.reshape(1, d), g2, b2,
        local_features, xi.reshape(1, 1))

    s = _sc_accum(n, d, k)(y, x, init1, src_p, dst_p)

    return _post_call(n, d)(
        s[:n], s[n:], out_w.T, out_b.reshape(1, d))
